# baseline (speedup 1.0000x reference)
import jax
import jax.numpy as jnp
from jax.experimental import pallas as pl
from jax.experimental.pallas import tpu as pltpu

_BP = 128


def _enc_kernel(pts_ref, pose_ref, goal_ref, W1_ref, b1_ref, W2_ref, b2_ref,
                W3a_ref, W3b_ref, b3_ref, W4_ref, b4_ref, Wg_ref, bg_ref,
                out_ref):
    pb = pl.program_id(1)
    T = 20
    BP = _BP

    x0 = pose_ref[0, :, 0:1]
    y0 = pose_ref[0, :, 1:2]
    th0 = pose_ref[0, :, 2:3]
    c = jnp.cos(th0)
    s = jnp.sin(th0)

    pts = pts_ref[0]
    dx = pts[:, 0:1] - x0
    dy = pts[:, 1:2] - y0
    lx = dx * c + dy * s
    ly = dy * c - dx * s
    lth = pts[:, 2:3] - th0

    h1 = jnp.maximum(
        lx * W1_ref[0:1, :] + ly * W1_ref[1:2, :] + lth * W1_ref[2:3, :]
        + b1_ref[:], 0.0)
    h = jnp.dot(h1, W2_ref[:], preferred_element_type=jnp.float32) + b2_ref[:]

    pooled = jnp.max(h.reshape(BP, T, 256), axis=1)

    a = jnp.dot(h, W3a_ref[:], preferred_element_type=jnp.float32)
    bpool = jnp.dot(pooled, W3b_ref[:],
                    preferred_element_type=jnp.float32) + b3_ref[:]
    o1 = jnp.maximum(a.reshape(BP, T, 256) + bpool[:, None, :], 0.0)
    o2 = jnp.dot(o1.reshape(BP * T, 256), W4_ref[:],
                 preferred_element_type=jnp.float32) + b4_ref[:]
    feat = jnp.max(o2.reshape(BP, T, 256), axis=1)
    part = jnp.sum(feat, axis=0, keepdims=True)

    @pl.when(pb == 0)
    def _init():
        gx = goal_ref[0, :, 0:1] - x0
        gy = goal_ref[0, :, 1:2] - y0
        glx = gx * c + gy * s
        gly = gy * c - gx * s
        glth = goal_ref[0, :, 2:3] - th0
        out_ref[0] = (glx * Wg_ref[0:1, :] + gly * Wg_ref[1:2, :]
                      + glth * Wg_ref[2:3, :] + bg_ref[:])

    out_ref[0] += part


def kernel(goal, pose, map_polylines, W1, b1, W2, b2, W3, b3, W4, b4, Wg, bg):
    B, M, P, T, C = map_polylines.shape
    D = Wg.shape[1]
    BM = B * M
    BP = _BP
    nb = P // BP

    pts = map_polylines.reshape(BM, P * T, C)
    pose2 = pose.reshape(BM, 1, C)
    goal2 = goal.reshape(BM, 1, C)
    W3a, W3b = W3[:256], W3[256:]
    b1r, b2r = b1.reshape(1, -1), b2.reshape(1, -1)
    b3r, b4r = b3.reshape(1, -1), b4.reshape(1, -1)
    bgr = bg.reshape(1, -1)

    wspec = lambda shape: pl.BlockSpec(shape, lambda bm, pb: (0, 0))
    out = pl.pallas_call(
        _enc_kernel,
        grid=(BM, nb),
        in_specs=[
            pl.BlockSpec((1, BP * T, C), lambda bm, pb: (bm, pb, 0)),
            pl.BlockSpec((1, 1, C), lambda bm, pb: (bm, 0, 0)),
            pl.BlockSpec((1, 1, C), lambda bm, pb: (bm, 0, 0)),
            wspec(W1.shape),
            wspec((1, b1.shape[0])),
            wspec(W2.shape),
            wspec((1, b2.shape[0])),
            wspec(W3a.shape),
            wspec(W3b.shape),
            wspec((1, b3.shape[0])),
            wspec(W4.shape),
            wspec((1, b4.shape[0])),
            wspec(Wg.shape),
            wspec((1, bg.shape[0])),
        ],
        out_specs=pl.BlockSpec((1, 1, D), lambda bm, pb: (bm, 0, 0)),
        out_shape=jax.ShapeDtypeStruct((BM, 1, D), jnp.float32),
        compiler_params=pltpu.CompilerParams(
            dimension_semantics=("parallel", "arbitrary")),
    )(pts, pose2, goal2, W1, b1r, W2, b2r, W3a, W3b, b3r, W4, b4r, Wg, bgr)
    return out.reshape(B, M, D)
